# baseline (speedup 1.0000x reference)
import jax
import jax.numpy as jnp
from jax.experimental import pallas as pl
from jax.experimental.pallas import tpu as pltpu

_TILE_N = 262144


def _body(xt_ref, zt_ref, w_ref, b_ref, o_ref):
    xz = jnp.concatenate([xt_ref[...], zt_ref[...]], axis=0)
    o_ref[...] = (
        jnp.dot(w_ref[...], xz, preferred_element_type=jnp.float32)
        + b_ref[...]
    )


def kernel(x, z, wxt, wzt, bt):
    B, depth = x.shape
    xt = x.T
    zt = z.T
    w_cat = jnp.concatenate([wxt, wzt], axis=1)

    grid = (pl.cdiv(B, _TILE_N),)
    in_spec = pl.BlockSpec((depth, _TILE_N), lambda i: (0, i))
    w_spec = pl.BlockSpec((depth, 2 * depth), lambda i: (0, 0))
    b_spec = pl.BlockSpec((depth, 1), lambda i: (0, 0))

    out_t = pl.pallas_call(
        _body,
        out_shape=jax.ShapeDtypeStruct((depth, B), jnp.float32),
        grid=grid,
        in_specs=[in_spec, in_spec, w_spec, b_spec],
        out_specs=in_spec,
        compiler_params=pltpu.CompilerParams(dimension_semantics=("parallel",)),
    )(xt, zt, w_cat, bt)

    return out_t.T


# speedup vs baseline: 1.0198x; 1.0198x over previous
import jax
import jax.numpy as jnp
from jax.experimental import pallas as pl
from jax.experimental.pallas import tpu as pltpu

_TILE_N = 131072


def _body(xt_ref, zt_ref, w_ref, b_ref, o_ref):
    xz = jnp.concatenate([xt_ref[...], zt_ref[...]], axis=0)
    o_ref[...] = (
        jnp.dot(w_ref[...], xz, preferred_element_type=jnp.float32)
        + b_ref[...]
    )


def kernel(x, z, wxt, wzt, bt):
    B, depth = x.shape
    xt = x.T
    zt = z.T
    w_cat = jnp.concatenate([wxt, wzt], axis=1)

    grid = (pl.cdiv(B, _TILE_N),)
    in_spec = pl.BlockSpec((depth, _TILE_N), lambda i: (0, i))
    w_spec = pl.BlockSpec((depth, 2 * depth), lambda i: (0, 0))
    b_spec = pl.BlockSpec((depth, 1), lambda i: (0, 0))

    out_t = pl.pallas_call(
        _body,
        out_shape=jax.ShapeDtypeStruct((depth, B), jnp.float32),
        grid=grid,
        in_specs=[in_spec, in_spec, w_spec, b_spec],
        out_specs=in_spec,
        compiler_params=pltpu.CompilerParams(dimension_semantics=("parallel",)),
    )(xt, zt, w_cat, bt)

    return out_t.T


# speedup vs baseline: 1.0724x; 1.0515x over previous
import jax
import jax.numpy as jnp
from jax.experimental import pallas as pl
from jax.experimental.pallas import tpu as pltpu

_TILE_N = 131072


def _body(xt_ref, zt_ref, wx_ref, wz_ref, b_ref, o_ref):
    o_ref[...] = (
        jnp.dot(wx_ref[...], xt_ref[...], preferred_element_type=jnp.float32)
        + jnp.dot(wz_ref[...], zt_ref[...], preferred_element_type=jnp.float32)
        + b_ref[...]
    )


def kernel(x, z, wxt, wzt, bt):
    B, depth = x.shape
    xt = x.T
    zt = z.T
    grid = (pl.cdiv(B, _TILE_N),)
    in_spec = pl.BlockSpec((depth, _TILE_N), lambda i: (0, i))
    w_spec = pl.BlockSpec((depth, depth), lambda i: (0, 0))
    b_spec = pl.BlockSpec((depth, 1), lambda i: (0, 0))

    out_t = pl.pallas_call(
        _body,
        out_shape=jax.ShapeDtypeStruct((depth, B), jnp.float32),
        grid=grid,
        in_specs=[in_spec, in_spec, w_spec, w_spec, b_spec],
        out_specs=in_spec,
        compiler_params=pltpu.CompilerParams(dimension_semantics=("parallel",)),
    )(xt, zt, wxt, wzt, bt)

    return out_t.T
